# baseline (speedup 1.0000x reference)
import jax
import jax.numpy as jnp
from jax.experimental import pallas as pl

_K = 16
_B = 4
_N = 4096
_R = 256
_NB = _N // _R
_INV_SIGMA = 1.0 / (0.75 * 0.75)
_S_SCALE = 8.0
_BIG = 3.0e38
_DENOM = float(_B * _N * _K)


def _eps_denom(x):
    s = jnp.sign(x)
    s = jnp.where(s == 0.0, 1.0, s)
    return s * jnp.maximum(jnp.abs(x), 1e-17)


def _dot(a, b, ca, cb, prec=None):
    return jax.lax.dot_general(
        a, b, (((ca,), (cb,)), ((), ())),
        preferred_element_type=jnp.float32, precision=prec)


def _two_sum(a, b):
    s = a + b
    bb = s - a
    return s, (a - (s - bb)) + (b - bb)


def _dist_block(PT, Pr):
    sqc = ((PT[0:1, :] * PT[0:1, :] + PT[1:2, :] * PT[1:2, :])
           + PT[2:3, :] * PT[2:3, :])
    sqr = ((Pr[:, 0:1] * Pr[:, 0:1] + Pr[:, 1:2] * Pr[:, 1:2])
           + Pr[:, 2:3] * Pr[:, 2:3])
    PTb = PT.astype(jnp.bfloat16).astype(jnp.float32)
    Prb = Pr.astype(jnp.bfloat16).astype(jnp.float32)
    px = Prb[:, 0:1] * PTb[0:1, :]
    py = Prb[:, 1:2] * PTb[1:2, :]
    pz = Prb[:, 2:3] * PTb[2:3, :]
    s1, e1 = _two_sum(px, py)
    s2, e2 = _two_sum(s1, pz)
    g = s2 + (e1 + e2)
    return (sqr + sqc) - 2.0 * g


def _pass_a_kernel(p_ref, n_ref, pt_ref, nt_ref, w_ref, ndt_ref):
    rb = pl.program_id(1)

    PT = pt_ref[0]
    NT = nt_ref[0]
    Pr = p_ref[0, pl.ds(rb * _R, _R), :]
    Nr = n_ref[0, pl.ds(rb * _R, _R), :]
    nut = NT / jnp.maximum(
        jnp.sqrt(jnp.sum(NT * NT, axis=0, keepdims=True)), 1e-12)
    nur = Nr / jnp.maximum(
        jnp.sqrt(jnp.sum(Nr * Nr, axis=1, keepdims=True)), 1e-12)

    d2 = _dist_block(PT, Pr)

    cur = d2
    m = jnp.min(cur, axis=1, keepdims=True)
    v1 = m
    cur = jnp.where(cur <= m, _BIG, cur)
    m = jnp.min(cur, axis=1, keepdims=True)
    d1 = m
    for _ in range(_K - 1):
        cur = jnp.where(cur <= m, _BIG, cur)
        m = jnp.min(cur, axis=1, keepdims=True)
    t17 = m

    rs = 1.0 / _eps_denom(d1 * _S_SCALE)
    phi = jnp.maximum(1.0 - d2 * rs, 0.0)
    phi = phi * phi
    phi = phi * phi
    dn = nut[0:1, :] - nur[:, 0:1]
    dsq = dn * dn
    dn = nut[1:2, :] - nur[:, 1:2]
    dsq = dsq + dn * dn
    dn = nut[2:3, :] - nur[:, 2:3]
    dsq = dsq + dn * dn
    nw = jnp.exp(-dsq * _INV_SIGMA)
    sel = jnp.logical_and(d2 > v1, d2 <= t17)
    w = jnp.where(sel, phi * nw, 0.0)

    w_ref[0] = w
    NT1 = jnp.concatenate((NT, jnp.ones((1, _N), jnp.float32)), axis=0)
    ndt_ref[0, :, pl.ds(rb * _R, _R)] = _dot(
        NT1, w, 1, 1, jax.lax.Precision.HIGHEST)


def _pass_b_kernel(p_ref, pt_ref, ndt_ref, w_ref, out_ref):
    b = pl.program_id(0)
    rb = pl.program_id(1)

    PT = pt_ref[0]
    Pr = p_ref[0, pl.ds(rb * _R, _R), :]
    w = w_ref[0]
    ndtw = ndt_ref[0]

    @pl.when(jnp.logical_and(b == 0, rb == 0))
    def _init():
        out_ref[:, :] = jnp.zeros((1, 1), jnp.float32)

    un = ndtw[0:3, :] / _eps_denom(ndtw[3:4, :])
    un = un / jnp.maximum(
        jnp.sqrt(jnp.sum(un * un, axis=0, keepdims=True)), 1e-12)
    dist = (PT[0:1, :] - Pr[:, 0:1]) * un[0:1, :]
    dist = dist + (PT[1:2, :] - Pr[:, 1:2]) * un[1:2, :]
    dist = dist + (PT[2:3, :] - Pr[:, 2:3]) * un[2:3, :]
    contrib = jnp.sum(dist * dist * w, axis=1, keepdims=True)
    contrib = jnp.sum(contrib, axis=0, keepdims=True) * (1.0 / _DENOM)
    out_ref[:, :] = out_ref[:, :] + contrib


def _surface_loss_pallas(points, normals):
    pt = jnp.transpose(points, (0, 2, 1))
    nt = jnp.transpose(normals, (0, 2, 1))
    w, ndt = pl.pallas_call(
        _pass_a_kernel,
        grid=(_B, _NB),
        in_specs=[
            pl.BlockSpec((1, _N, 3), lambda b, rb: (b, 0, 0)),
            pl.BlockSpec((1, _N, 3), lambda b, rb: (b, 0, 0)),
            pl.BlockSpec((1, 3, _N), lambda b, rb: (b, 0, 0)),
            pl.BlockSpec((1, 3, _N), lambda b, rb: (b, 0, 0)),
        ],
        out_specs=[
            pl.BlockSpec((1, _R, _N), lambda b, rb: (b, rb, 0)),
            pl.BlockSpec((1, 4, _N), lambda b, rb: (b, 0, 0)),
        ],
        out_shape=[
            jax.ShapeDtypeStruct((_B, _N, _N), jnp.float32),
            jax.ShapeDtypeStruct((_B, 4, _N), jnp.float32),
        ],
    )(points, normals, pt, nt)
    out = pl.pallas_call(
        _pass_b_kernel,
        grid=(_B, _NB),
        in_specs=[
            pl.BlockSpec((1, _N, 3), lambda b, rb: (b, 0, 0)),
            pl.BlockSpec((1, 3, _N), lambda b, rb: (b, 0, 0)),
            pl.BlockSpec((1, 4, _N), lambda b, rb: (b, 0, 0)),
            pl.BlockSpec((1, _R, _N), lambda b, rb: (b, rb, 0)),
        ],
        out_specs=pl.BlockSpec((1, 1), lambda b, rb: (0, 0)),
        out_shape=jax.ShapeDtypeStruct((1, 1), jnp.float32),
    )(points, pt, ndt, w)
    return out[0, 0]


def kernel(points, normals):
    return _surface_loss_pallas(points, normals)


# speedup vs baseline: 1.1858x; 1.1858x over previous
import jax
import jax.numpy as jnp
from jax.experimental import pallas as pl

_K = 16
_B = 4
_N = 4096
_R = 256
_NB = _N // _R
_INV_SIGMA = 1.0 / (0.75 * 0.75)
_S_SCALE = 8.0
_BIG = 3.0e38
_DENOM = float(_B * _N * _K)


def _eps_denom(x):
    s = jnp.sign(x)
    s = jnp.where(s == 0.0, 1.0, s)
    return s * jnp.maximum(jnp.abs(x), 1e-17)


def _dot(a, b, ca, cb, prec=None):
    return jax.lax.dot_general(
        a, b, (((ca,), (cb,)), ((), ())),
        preferred_element_type=jnp.float32, precision=prec)


def _two_sum(a, b):
    s = a + b
    bb = s - a
    return s, (a - (s - bb)) + (b - bb)


def _dist_block(PT, Pr):
    sqc = ((PT[0:1, :] * PT[0:1, :] + PT[1:2, :] * PT[1:2, :])
           + PT[2:3, :] * PT[2:3, :])
    sqr = ((Pr[:, 0:1] * Pr[:, 0:1] + Pr[:, 1:2] * Pr[:, 1:2])
           + Pr[:, 2:3] * Pr[:, 2:3])
    g = jax.lax.dot_general(
        Pr.astype(jnp.bfloat16), PT.astype(jnp.bfloat16),
        (((1,), (0,)), ((), ())), preferred_element_type=jnp.float32)
    return (sqr + sqc) - 2.0 * g


def _pass_a_kernel(p_ref, n_ref, pt_ref, nt_ref, w_ref, ndt_ref):
    rb = pl.program_id(1)

    PT = pt_ref[0]
    NT = nt_ref[0]
    Pr = p_ref[0, pl.ds(rb * _R, _R), :]
    Nr = n_ref[0, pl.ds(rb * _R, _R), :]
    nut = NT / jnp.maximum(
        jnp.sqrt(jnp.sum(NT * NT, axis=0, keepdims=True)), 1e-12)
    nur = Nr / jnp.maximum(
        jnp.sqrt(jnp.sum(Nr * Nr, axis=1, keepdims=True)), 1e-12)

    d2 = _dist_block(PT, Pr)

    cur = d2
    m = jnp.min(cur, axis=1, keepdims=True)
    v1 = m
    cur = jnp.where(cur <= m, _BIG, cur)
    m = jnp.min(cur, axis=1, keepdims=True)
    d1 = m
    for _ in range(_K - 1):
        cur = jnp.where(cur <= m, _BIG, cur)
        m = jnp.min(cur, axis=1, keepdims=True)
    t17 = m

    rs = 1.0 / _eps_denom(d1 * _S_SCALE)
    phi = jnp.maximum(1.0 - d2 * rs, 0.0)
    phi = phi * phi
    phi = phi * phi
    dn = nut[0:1, :] - nur[:, 0:1]
    dsq = dn * dn
    dn = nut[1:2, :] - nur[:, 1:2]
    dsq = dsq + dn * dn
    dn = nut[2:3, :] - nur[:, 2:3]
    dsq = dsq + dn * dn
    nw = jnp.exp(-dsq * _INV_SIGMA)
    sel = jnp.logical_and(d2 > v1, d2 <= t17)
    w = jnp.where(sel, phi * nw, 0.0)

    w_ref[0] = w
    NT1 = jnp.concatenate((NT, jnp.ones((1, _N), jnp.float32)), axis=0)
    ndt_ref[0, :, pl.ds(rb * _R, _R)] = _dot(
        NT1, w, 1, 1, jax.lax.Precision.HIGHEST)


def _pass_b_kernel(p_ref, pt_ref, ndt_ref, w_ref, out_ref):
    b = pl.program_id(0)
    rb = pl.program_id(1)

    PT = pt_ref[0]
    Pr = p_ref[0, pl.ds(rb * _R, _R), :]
    w = w_ref[0]
    ndtw = ndt_ref[0]

    @pl.when(jnp.logical_and(b == 0, rb == 0))
    def _init():
        out_ref[:, :] = jnp.zeros((1, 1), jnp.float32)

    un = ndtw[0:3, :] / _eps_denom(ndtw[3:4, :])
    un = un / jnp.maximum(
        jnp.sqrt(jnp.sum(un * un, axis=0, keepdims=True)), 1e-12)
    dist = (PT[0:1, :] - Pr[:, 0:1]) * un[0:1, :]
    dist = dist + (PT[1:2, :] - Pr[:, 1:2]) * un[1:2, :]
    dist = dist + (PT[2:3, :] - Pr[:, 2:3]) * un[2:3, :]
    contrib = jnp.sum(dist * dist * w, axis=1, keepdims=True)
    contrib = jnp.sum(contrib, axis=0, keepdims=True) * (1.0 / _DENOM)
    out_ref[:, :] = out_ref[:, :] + contrib


def _surface_loss_pallas(points, normals):
    pt = jnp.transpose(points, (0, 2, 1))
    nt = jnp.transpose(normals, (0, 2, 1))
    w, ndt = pl.pallas_call(
        _pass_a_kernel,
        grid=(_B, _NB),
        in_specs=[
            pl.BlockSpec((1, _N, 3), lambda b, rb: (b, 0, 0)),
            pl.BlockSpec((1, _N, 3), lambda b, rb: (b, 0, 0)),
            pl.BlockSpec((1, 3, _N), lambda b, rb: (b, 0, 0)),
            pl.BlockSpec((1, 3, _N), lambda b, rb: (b, 0, 0)),
        ],
        out_specs=[
            pl.BlockSpec((1, _R, _N), lambda b, rb: (b, rb, 0)),
            pl.BlockSpec((1, 4, _N), lambda b, rb: (b, 0, 0)),
        ],
        out_shape=[
            jax.ShapeDtypeStruct((_B, _N, _N), jnp.float32),
            jax.ShapeDtypeStruct((_B, 4, _N), jnp.float32),
        ],
    )(points, normals, pt, nt)
    out = pl.pallas_call(
        _pass_b_kernel,
        grid=(_B, _NB),
        in_specs=[
            pl.BlockSpec((1, _N, 3), lambda b, rb: (b, 0, 0)),
            pl.BlockSpec((1, 3, _N), lambda b, rb: (b, 0, 0)),
            pl.BlockSpec((1, 4, _N), lambda b, rb: (b, 0, 0)),
            pl.BlockSpec((1, _R, _N), lambda b, rb: (b, rb, 0)),
        ],
        out_specs=pl.BlockSpec((1, 1), lambda b, rb: (0, 0)),
        out_shape=jax.ShapeDtypeStruct((1, 1), jnp.float32),
    )(points, pt, ndt, w)
    return out[0, 0]


def kernel(points, normals):
    return _surface_loss_pallas(points, normals)
